# baseline (speedup 1.0000x reference)
import functools

import numpy as np
import jax
import jax.numpy as jnp
from jax.experimental import pallas as pl
from jax.experimental.pallas import tpu as pltpu
from jax.experimental.pallas import tpu_sc as plsc

D_MODEL = 16
N_FIELDS = 26
TOTAL = D_MODEL * N_FIELDS

_GATHER_WINDOW = 2048
_ROW_BLK = 1280


def _gather_rows(embed, idx_flat, n_idx):
    mesh = plsc.VectorSubcoreMesh(core_axis_name="core", subcore_axis_name="subcore")

    @functools.partial(
        pl.kernel,
        out_type=jax.ShapeDtypeStruct((n_idx, D_MODEL), embed.dtype),
        mesh=mesh,
        compiler_params=pltpu.CompilerParams(use_tc_tiling_on_sc=False),
    )
    def gather_kernel(table_hbm, i_hbm, o_hbm):
        def body(i_vmem, o_vmem):
            pltpu.sync_copy(table_hbm.at[i_vmem], o_vmem)

        pltpu.emit_pipeline(
            body,
            grid=(n_idx // _GATHER_WINDOW,),
            in_specs=[
                pl.BlockSpec((_GATHER_WINDOW,), index_map=lambda i: (i,))
            ],
            out_specs=[
                pl.BlockSpec((_GATHER_WINDOW, D_MODEL), index_map=lambda i: (i, 0))
            ],
            core_axis_name=("core", "subcore"),
            dimension_semantics=(pltpu.PARALLEL,),
        )(i_hbm, o_hbm)

    return gather_kernel(embed, idx_flat)


_PAD_FIELDS = 32
_ROW_F32 = _PAD_FIELDS * D_MODEL


def _mlp_body(x_ref, w1_ref, b1_ref, w2_ref, peb2_ref, o_ref):
    x = x_ref[...].reshape(_ROW_BLK, _ROW_F32).astype(jnp.bfloat16)
    h = jnp.dot(x, w1_ref[...], preferred_element_type=jnp.float32)
    h = h + b1_ref[...]
    t = jnp.exp(jnp.minimum(h, 20.0).astype(jnp.bfloat16))
    u = t * (t + jnp.bfloat16(2.0))
    m = h * (u / (u + jnp.bfloat16(2.0))).astype(jnp.float32)
    o_ref[...] = (
        jnp.dot(m.astype(jnp.bfloat16), w2_ref[...], preferred_element_type=jnp.float32)
        + peb2_ref[...]
    )


def _pe_tile(hist, rows):
    pos = np.arange(hist, dtype=np.float32)[:, None]
    div = np.exp(
        np.arange(0, D_MODEL, 2, dtype=np.float32) * (-np.log(10000.0) / D_MODEL)
    )
    pe = np.zeros((hist, D_MODEL), dtype=np.float32)
    pe[:, 0::2] = np.sin(pos * div)
    pe[:, 1::2] = np.cos(pos * div)
    return np.tile(pe, (rows // hist, 1))


def kernel(events, embed, W1, b1, W2, b2):
    batch, hist, n_fields = events.shape
    n_rows = batch * hist
    n_idx = n_rows * n_fields

    n_idx_pad = n_rows * _PAD_FIELDS
    idx_pad = jnp.pad(
        events.reshape(n_rows, n_fields), ((0, 0), (0, _PAD_FIELDS - n_fields))
    ).reshape(n_idx_pad)
    gathered = _gather_rows(embed, idx_pad, n_idx_pad)
    x128 = gathered.reshape(n_rows * _ROW_F32 // 128, 128)

    w1b = jnp.pad(W1, ((0, _ROW_F32 - TOTAL), (0, 0))).astype(jnp.bfloat16)
    w2b = W2.astype(jnp.bfloat16)
    b1r = b1.reshape(1, TOTAL * 4)
    peb2 = jnp.asarray(_pe_tile(hist, _ROW_BLK)) + b2[None, :]

    blk128 = _ROW_BLK * _ROW_F32 // 128
    out = pl.pallas_call(
        _mlp_body,
        grid=(n_rows // _ROW_BLK,),
        in_specs=[
            pl.BlockSpec((blk128, 128), lambda i: (i, 0)),
            pl.BlockSpec((_ROW_F32, TOTAL * 4), lambda i: (0, 0)),
            pl.BlockSpec((1, TOTAL * 4), lambda i: (0, 0)),
            pl.BlockSpec((TOTAL * 4, D_MODEL), lambda i: (0, 0)),
            pl.BlockSpec((_ROW_BLK, D_MODEL), lambda i: (0, 0)),
        ],
        out_specs=pl.BlockSpec((_ROW_BLK, D_MODEL), lambda i: (i, 0)),
        out_shape=jax.ShapeDtypeStruct((n_rows, D_MODEL), jnp.float32),
    )(x128, w1b, b1r, w2b, peb2)

    return out.reshape(batch, hist, D_MODEL)


# speedup vs baseline: 3.0178x; 3.0178x over previous
import functools

import numpy as np
import jax
import jax.numpy as jnp
from jax.experimental import pallas as pl
from jax.experimental.pallas import tpu as pltpu
from jax.experimental.pallas import tpu_sc as plsc

D_MODEL = 16
N_FIELDS = 26
TOTAL = D_MODEL * N_FIELDS

_GATHER_WINDOW = 2048
_ROW_BLK = 1280


def _gather_rows(embed, idx_flat, n_idx):
    mesh = plsc.VectorSubcoreMesh(core_axis_name="core", subcore_axis_name="subcore")

    @functools.partial(
        pl.kernel,
        out_type=jax.ShapeDtypeStruct((n_idx, D_MODEL), embed.dtype),
        mesh=mesh,
        compiler_params=pltpu.CompilerParams(use_tc_tiling_on_sc=False),
    )
    def gather_kernel(table_hbm, i_hbm, o_hbm):
        def body(i_vmem, o_vmem):
            pltpu.sync_copy(table_hbm.at[i_vmem], o_vmem)

        pltpu.emit_pipeline(
            body,
            grid=(n_idx // _GATHER_WINDOW,),
            in_specs=[
                pl.BlockSpec((_GATHER_WINDOW,), index_map=lambda i: (i,))
            ],
            out_specs=[
                pl.BlockSpec((_GATHER_WINDOW, D_MODEL), index_map=lambda i: (i, 0))
            ],
            core_axis_name=("core", "subcore"),
            dimension_semantics=(pltpu.PARALLEL,),
        )(i_hbm, o_hbm)

    return gather_kernel(embed, idx_flat)


def _mlp_body(x_ref, w1_ref, b1_ref, w2_ref, peb2_ref, o_ref):
    x = x_ref[...].astype(jnp.bfloat16)
    h = jnp.dot(x, w1_ref[...], preferred_element_type=jnp.float32)
    h = h + b1_ref[...]
    t = jnp.exp(jnp.minimum(h, 20.0).astype(jnp.bfloat16))
    u = t * (t + jnp.bfloat16(2.0))
    m = h * (u / (u + jnp.bfloat16(2.0))).astype(jnp.float32)
    o_ref[...] = (
        jnp.dot(m.astype(jnp.bfloat16), w2_ref[...], preferred_element_type=jnp.float32)
        + peb2_ref[...]
    )


def _pe_tile(hist, rows):
    pos = np.arange(hist, dtype=np.float32)[:, None]
    div = np.exp(
        np.arange(0, D_MODEL, 2, dtype=np.float32) * (-np.log(10000.0) / D_MODEL)
    )
    pe = np.zeros((hist, D_MODEL), dtype=np.float32)
    pe[:, 0::2] = np.sin(pos * div)
    pe[:, 1::2] = np.cos(pos * div)
    return np.tile(pe, (rows // hist, 1))


def kernel(events, embed, W1, b1, W2, b2):
    batch, hist, n_fields = events.shape
    n_rows = batch * hist
    n_idx = n_rows * n_fields

    w1b = W1.astype(jnp.bfloat16)
    w2b = W2.astype(jnp.bfloat16)
    b1r = b1.reshape(1, TOTAL * 4)
    peb2 = jnp.asarray(_pe_tile(hist, _ROW_BLK)) + b2[None, :]

    n_chunks = 4
    rows_c = n_rows // n_chunks
    idx_c = rows_c * n_fields
    idx_flat = events.reshape(n_chunks, idx_c)
    outs = []
    for k in range(n_chunks):
        idx_k = jax.lax.squeeze(jax.lax.slice_in_dim(idx_flat, k, k + 1), (0,))
        g_k = _gather_rows(embed, idx_k, idx_c)
        x_k = g_k.reshape(rows_c, TOTAL)
        out_k = pl.pallas_call(
            _mlp_body,
            grid=(rows_c // _ROW_BLK,),
            in_specs=[
                pl.BlockSpec((_ROW_BLK, TOTAL), lambda i: (i, 0)),
                pl.BlockSpec((TOTAL, TOTAL * 4), lambda i: (0, 0)),
                pl.BlockSpec((1, TOTAL * 4), lambda i: (0, 0)),
                pl.BlockSpec((TOTAL * 4, D_MODEL), lambda i: (0, 0)),
                pl.BlockSpec((_ROW_BLK, D_MODEL), lambda i: (0, 0)),
            ],
            out_specs=pl.BlockSpec((_ROW_BLK, D_MODEL), lambda i: (i, 0)),
            out_shape=jax.ShapeDtypeStruct((rows_c, D_MODEL), jnp.float32),
        )(x_k, w1b, b1r, w2b, peb2)
        outs.append(out_k)

    out = jnp.concatenate(outs, axis=0)
    return out.reshape(batch, hist, D_MODEL)


# speedup vs baseline: 3.3653x; 1.1151x over previous
import functools

import numpy as np
import jax
import jax.numpy as jnp
from jax.experimental import pallas as pl
from jax.experimental.pallas import tpu as pltpu
from jax.experimental.pallas import tpu_sc as plsc

D_MODEL = 16
N_FIELDS = 26
TOTAL = D_MODEL * N_FIELDS

_GATHER_WINDOW = 2048
_ROW_BLK = 1280


def _gather_rows(embed, idx_flat, n_idx):
    mesh = plsc.VectorSubcoreMesh(core_axis_name="core", subcore_axis_name="subcore")

    @functools.partial(
        pl.kernel,
        out_type=jax.ShapeDtypeStruct((n_idx, D_MODEL), embed.dtype),
        mesh=mesh,
        compiler_params=pltpu.CompilerParams(use_tc_tiling_on_sc=False),
    )
    def gather_kernel(table_hbm, i_hbm, o_hbm):
        def body(i_vmem, o_vmem):
            pltpu.sync_copy(table_hbm.at[i_vmem], o_vmem)

        pltpu.emit_pipeline(
            body,
            grid=(n_idx // _GATHER_WINDOW,),
            in_specs=[
                pl.BlockSpec((_GATHER_WINDOW,), index_map=lambda i: (i,))
            ],
            out_specs=[
                pl.BlockSpec((_GATHER_WINDOW, D_MODEL), index_map=lambda i: (i, 0))
            ],
            core_axis_name=("core", "subcore"),
            dimension_semantics=(pltpu.PARALLEL,),
        )(i_hbm, o_hbm)

    return gather_kernel(embed, idx_flat)


_PAD_FIELDS = 32
_ROW_F32 = _PAD_FIELDS * D_MODEL


def _mlp_body(x_ref, w1_ref, b1_ref, w2_ref, peb2_ref, o_ref):
    x = x_ref[...].reshape(_ROW_BLK, _ROW_F32).astype(jnp.bfloat16)
    h = jnp.dot(x, w1_ref[...], preferred_element_type=jnp.float32)
    h = h + b1_ref[...]
    t = jnp.exp(jnp.minimum(h, 20.0).astype(jnp.bfloat16))
    u = t * (t + jnp.bfloat16(2.0))
    m = h * (u / (u + jnp.bfloat16(2.0))).astype(jnp.float32)
    o_ref[...] = (
        jnp.dot(m.astype(jnp.bfloat16), w2_ref[...], preferred_element_type=jnp.float32)
        + peb2_ref[...]
    )


def _pe_tile(hist, rows):
    pos = np.arange(hist, dtype=np.float32)[:, None]
    div = np.exp(
        np.arange(0, D_MODEL, 2, dtype=np.float32) * (-np.log(10000.0) / D_MODEL)
    )
    pe = np.zeros((hist, D_MODEL), dtype=np.float32)
    pe[:, 0::2] = np.sin(pos * div)
    pe[:, 1::2] = np.cos(pos * div)
    return np.tile(pe, (rows // hist, 1))


def kernel(events, embed, W1, b1, W2, b2):
    batch, hist, n_fields = events.shape
    n_rows = batch * hist
    n_idx = n_rows * n_fields

    w1b = jnp.pad(W1, ((0, _ROW_F32 - TOTAL), (0, 0))).astype(jnp.bfloat16)
    w2b = W2.astype(jnp.bfloat16)
    b1r = b1.reshape(1, TOTAL * 4)
    peb2 = jnp.asarray(_pe_tile(hist, _ROW_BLK)) + b2[None, :]
    blk128 = _ROW_BLK * _ROW_F32 // 128

    n_chunks = 4
    rows_c = n_rows // n_chunks
    events_2d = events.reshape(n_rows, n_fields)
    outs = []
    for k in range(n_chunks):
        ev_k = jax.lax.slice_in_dim(events_2d, k * rows_c, (k + 1) * rows_c)
        idx_k = jnp.concatenate(
            [ev_k, jax.lax.slice_in_dim(ev_k, 0, _PAD_FIELDS - n_fields, axis=1)],
            axis=1,
        ).reshape(rows_c * _PAD_FIELDS)
        g_k = _gather_rows(embed, idx_k, rows_c * _PAD_FIELDS)
        x_k = g_k.reshape(rows_c * _ROW_F32 // 128, 128)
        out_k = pl.pallas_call(
            _mlp_body,
            grid=(rows_c // _ROW_BLK,),
            in_specs=[
                pl.BlockSpec((blk128, 128), lambda i: (i, 0)),
                pl.BlockSpec((_ROW_F32, TOTAL * 4), lambda i: (0, 0)),
                pl.BlockSpec((1, TOTAL * 4), lambda i: (0, 0)),
                pl.BlockSpec((TOTAL * 4, D_MODEL), lambda i: (0, 0)),
                pl.BlockSpec((_ROW_BLK, D_MODEL), lambda i: (0, 0)),
            ],
            out_specs=pl.BlockSpec((_ROW_BLK, D_MODEL), lambda i: (i, 0)),
            out_shape=jax.ShapeDtypeStruct((rows_c, D_MODEL), jnp.float32),
        )(x_k, w1b, b1r, w2b, peb2)
        outs.append(out_k)

    out = jnp.concatenate(outs, axis=0)
    return out.reshape(batch, hist, D_MODEL)


# speedup vs baseline: 3.4033x; 1.0113x over previous
import functools

import numpy as np
import jax
import jax.numpy as jnp
from jax.experimental import pallas as pl
from jax.experimental.pallas import tpu as pltpu
from jax.experimental.pallas import tpu_sc as plsc

D_MODEL = 16
N_FIELDS = 26
TOTAL = D_MODEL * N_FIELDS

_GATHER_WINDOW = 2048
_ROW_BLK = 1280


def _gather_rows(embed, idx_flat, n_idx):
    mesh = plsc.VectorSubcoreMesh(core_axis_name="core", subcore_axis_name="subcore")

    @functools.partial(
        pl.kernel,
        out_type=jax.ShapeDtypeStruct((n_idx, D_MODEL), embed.dtype),
        mesh=mesh,
        compiler_params=pltpu.CompilerParams(use_tc_tiling_on_sc=False),
    )
    def gather_kernel(table_hbm, i_hbm, o_hbm):
        def body(i_vmem, o_vmem):
            pltpu.sync_copy(table_hbm.at[i_vmem], o_vmem)

        pltpu.emit_pipeline(
            body,
            grid=(n_idx // _GATHER_WINDOW,),
            in_specs=[
                pl.BlockSpec((_GATHER_WINDOW,), index_map=lambda i: (i,))
            ],
            out_specs=[
                pl.BlockSpec((_GATHER_WINDOW, D_MODEL), index_map=lambda i: (i, 0))
            ],
            core_axis_name=("core", "subcore"),
            dimension_semantics=(pltpu.PARALLEL,),
        )(i_hbm, o_hbm)

    return gather_kernel(embed, idx_flat)


_PAD_FIELDS = 32
_ROW_F32 = _PAD_FIELDS * D_MODEL


def _mlp_body(x_ref, w1_ref, b1_ref, w2_ref, peb2_ref, o_ref):
    x = x_ref[...].reshape(_ROW_BLK, _ROW_F32).astype(jnp.bfloat16)
    h = jnp.dot(x, w1_ref[...], preferred_element_type=jnp.float32)
    h = h + b1_ref[...]
    t = jnp.exp(jnp.minimum(h, 20.0).astype(jnp.bfloat16))
    u = t * (t + jnp.bfloat16(2.0))
    m = h * (u / (u + jnp.bfloat16(2.0))).astype(jnp.float32)
    o_ref[...] = (
        jnp.dot(m.astype(jnp.bfloat16), w2_ref[...], preferred_element_type=jnp.float32)
        + peb2_ref[...]
    )


def _pe_tile(hist, rows):
    pos = np.arange(hist, dtype=np.float32)[:, None]
    div = np.exp(
        np.arange(0, D_MODEL, 2, dtype=np.float32) * (-np.log(10000.0) / D_MODEL)
    )
    pe = np.zeros((hist, D_MODEL), dtype=np.float32)
    pe[:, 0::2] = np.sin(pos * div)
    pe[:, 1::2] = np.cos(pos * div)
    return np.tile(pe, (rows // hist, 1))


def kernel(events, embed, W1, b1, W2, b2):
    batch, hist, n_fields = events.shape
    n_rows = batch * hist
    n_idx = n_rows * n_fields

    w1b = jnp.pad(W1, ((0, _ROW_F32 - TOTAL), (0, 0))).astype(jnp.bfloat16)
    w2b = W2.astype(jnp.bfloat16)
    b1r = b1.reshape(1, TOTAL * 4)
    peb2 = jnp.asarray(_pe_tile(hist, _ROW_BLK)) + b2[None, :]
    blk128 = _ROW_BLK * _ROW_F32 // 128

    n_chunks = 8
    rows_c = n_rows // n_chunks
    events_2d = events.reshape(n_rows, n_fields)
    col_iota = jax.lax.broadcasted_iota(jnp.int32, (rows_c, _PAD_FIELDS), 1)
    row_iota = jax.lax.broadcasted_iota(jnp.int32, (rows_c, _PAD_FIELDS), 0)
    dummy = row_iota % embed.shape[0]
    outs = []
    for k in range(n_chunks):
        ev_k = jax.lax.slice_in_dim(events_2d, k * rows_c, (k + 1) * rows_c)
        idx_k = jnp.where(
            col_iota < n_fields,
            jnp.pad(ev_k, ((0, 0), (0, _PAD_FIELDS - n_fields))),
            dummy,
        ).reshape(rows_c * _PAD_FIELDS)
        g_k = _gather_rows(embed, idx_k, rows_c * _PAD_FIELDS)
        x_k = g_k.reshape(rows_c * _ROW_F32 // 128, 128)
        out_k = pl.pallas_call(
            _mlp_body,
            grid=(rows_c // _ROW_BLK,),
            in_specs=[
                pl.BlockSpec((blk128, 128), lambda i: (i, 0)),
                pl.BlockSpec((_ROW_F32, TOTAL * 4), lambda i: (0, 0)),
                pl.BlockSpec((1, TOTAL * 4), lambda i: (0, 0)),
                pl.BlockSpec((TOTAL * 4, D_MODEL), lambda i: (0, 0)),
                pl.BlockSpec((_ROW_BLK, D_MODEL), lambda i: (0, 0)),
            ],
            out_specs=pl.BlockSpec((_ROW_BLK, D_MODEL), lambda i: (i, 0)),
            out_shape=jax.ShapeDtypeStruct((rows_c, D_MODEL), jnp.float32),
        )(x_k, w1b, b1r, w2b, peb2)
        outs.append(out_k)

    out = jnp.concatenate(outs, axis=0)
    return out.reshape(batch, hist, D_MODEL)
